# Initial kernel scaffold; baseline (speedup 1.0000x reference)
import jax
import jax.numpy as jnp
from jax.experimental import pallas as pl


def kernel(hidden_states, hash_proj, W1, b1, W2, b2):
    raise NotImplementedError("write your pallas kernel here")


# speedup vs baseline: 1.0525x; 1.0525x over previous
import functools

import jax
import jax.numpy as jnp
from jax import lax
from jax.experimental import pallas as pl
from jax.experimental.pallas import tpu as pltpu
from jax.experimental.pallas import tpu_sc as plsc

_K = 2
_CAP = 2
_NC = 2
_NS = 16
_NW = _NC * _NS
_LANES = 16


def _sc_mesh():
    return plsc.VectorSubcoreMesh(
        core_axis_name="c", subcore_axis_name="s", num_cores=_NC,
        num_subcores=_NS)


def _wid():
    return lax.axis_index("s") * _NC + lax.axis_index("c")


def _dispatch_gather(x, gsrc, n_rows, d):
    rows_w = n_rows // _NW
    chunk = min(rows_w, 128)
    n_chunks = rows_w // chunk

    @functools.partial(
        pl.kernel,
        out_type=jax.ShapeDtypeStruct((n_rows, d), jnp.float32),
        mesh=_sc_mesh(),
        scratch_types=[
            pltpu.VMEM((chunk,), jnp.int32),
            pltpu.VMEM((chunk, d), jnp.float32),
            pltpu.SemaphoreType.DMA,
        ],
    )
    def k(x_hbm, gsrc_hbm, out_hbm, idx_v, buf_v, sem):
        base = _wid() * rows_w
        for i in range(n_chunks):
            off = base + i * chunk
            pltpu.sync_copy(gsrc_hbm.at[pl.ds(off, chunk)], idx_v)
            pltpu.async_copy(x_hbm.at[idx_v], buf_v, sem).wait()
            pltpu.sync_copy(buf_v, out_hbm.at[pl.ds(off, chunk)])

    return k(x, gsrc)


def _combine(y2d, cidx0, cidx1, t, d):
    tok_w = t // _NW
    nvc = d // _LANES

    @functools.partial(
        pl.kernel,
        out_type=jax.ShapeDtypeStruct((t, d), jnp.float32),
        mesh=_sc_mesh(),
        scratch_types=[
            pltpu.VMEM((tok_w,), jnp.int32),
            pltpu.VMEM((tok_w,), jnp.int32),
            pltpu.VMEM((tok_w, d), jnp.float32),
            pltpu.VMEM((tok_w, d), jnp.float32),
            pltpu.SemaphoreType.DMA,
            pltpu.SemaphoreType.DMA,
        ],
    )
    def k(y_hbm, c0_hbm, c1_hbm, out_hbm, i0_v, i1_v, g0_v, g1_v, s0, s1):
        base = _wid() * tok_w
        pltpu.sync_copy(c0_hbm.at[pl.ds(base, tok_w)], i0_v)
        cp0 = pltpu.async_copy(y_hbm.at[i0_v], g0_v, s0)
        pltpu.sync_copy(c1_hbm.at[pl.ds(base, tok_w)], i1_v)
        cp1 = pltpu.async_copy(y_hbm.at[i1_v], g1_v, s1)
        cp0.wait()
        cp1.wait()

        def row_add(r, carry):
            for u in range(nvc):
                sl = pl.ds(u * _LANES, _LANES)
                g0_v[r, sl] = g0_v[r, sl] + g1_v[r, sl]
            return carry

        lax.fori_loop(0, tok_w, row_add, 0)
        pltpu.sync_copy(g0_v, out_hbm.at[pl.ds(base, tok_w)])

    return k(y2d, cidx0, cidx1)


def _ffn_body(n_e, n_f, src_ref, w1_ref, b1_ref, w2_ref, b2_ref, m_ref,
              y_ref, acc_ref):
    e = pl.program_id(0)
    fi = pl.program_id(1)

    @pl.when(e < n_e)
    def _compute():
        xe = src_ref[0].astype(jnp.bfloat16)
        hm = jnp.dot(xe, w1_ref[0].astype(jnp.bfloat16),
                     preferred_element_type=jnp.float32)
        hm = jax.nn.gelu(hm + b1_ref[0, 0, :])
        part = jnp.dot(hm.astype(jnp.bfloat16), w2_ref[0].astype(jnp.bfloat16),
                       preferred_element_type=jnp.float32)

        @pl.when(fi == 0)
        def _():
            acc_ref[...] = part

        @pl.when(fi > 0)
        def _():
            acc_ref[...] += part

        @pl.when(fi == n_f - 1)
        def _():
            y_ref[0] = (acc_ref[...] + b2_ref[0, 0, :]) * m_ref[0]

    @pl.when(e == n_e)
    def _pad_zero():
        y_ref[0] = jnp.zeros_like(y_ref[0])


def _expert_ffn(xe3, W1, b1r, W2, b2r, vmask, n_e, c, d, f):
    n_f = 2
    fb = f // n_f
    grid = (n_e + 1, n_f)
    clamp = lambda e: jnp.minimum(e, n_e - 1)
    return pl.pallas_call(
        functools.partial(_ffn_body, n_e, n_f),
        grid=grid,
        in_specs=[
            pl.BlockSpec((1, c, d), lambda e, fi: (clamp(e), 0, 0)),
            pl.BlockSpec((1, d, fb), lambda e, fi: (clamp(e), 0, fi)),
            pl.BlockSpec((1, 1, fb), lambda e, fi: (clamp(e), 0, fi)),
            pl.BlockSpec((1, fb, d), lambda e, fi: (clamp(e), fi, 0)),
            pl.BlockSpec((1, 1, d), lambda e, fi: (clamp(e), 0, 0)),
            pl.BlockSpec((1, c, 1), lambda e, fi: (clamp(e), 0, 0)),
        ],
        out_specs=pl.BlockSpec((1, c, d), lambda e, fi: (e, 0, 0)),
        out_shape=jax.ShapeDtypeStruct((n_e + 1, c, d), jnp.float32),
        scratch_shapes=[pltpu.VMEM((c, d), jnp.float32)],
    )(xe3, W1, b1r, W2, b2r, vmask)


def kernel(hidden_states, hash_proj, W1, b1, W2, b2):
    B, S, D = hidden_states.shape
    E, _, F = W1.shape
    T = B * S
    C = _CAP * (-(-(T * _K) // E))

    x = hidden_states.reshape(T, D)

    h = x @ hash_proj.T
    buckets = jnp.mod(jnp.floor(jnp.abs(h) * 997.0).astype(jnp.int32), E)
    flat_e = buckets[:, :_K].reshape(-1)

    order = jnp.argsort(flat_e, stable=True).astype(jnp.int32)
    counts = jnp.bincount(flat_e, length=E).astype(jnp.int32)
    starts = (jnp.cumsum(counts) - counts).astype(jnp.int32)
    cpos = jnp.arange(C, dtype=jnp.int32)
    slot_j = order[jnp.clip(starts[:, None] + cpos[None, :], 0, T * _K - 1)]
    valid = cpos[None, :] < jnp.minimum(counts, C)[:, None]
    gsrc = jnp.where(valid, slot_j // _K, 0).reshape(-1).astype(jnp.int32)
    vmask = jnp.where(valid, 0.5, 0.0).astype(jnp.float32).reshape(E, C, 1)

    e_sorted = flat_e[order]
    pos_sorted = jnp.arange(T * _K, dtype=jnp.int32) - starts[e_sorted]
    pos = jnp.zeros((T * _K,), jnp.int32).at[order].set(pos_sorted)
    cidx = jnp.where(pos < C, flat_e * C + pos, E * C).astype(jnp.int32)
    cidx0 = cidx[0::_K]
    cidx1 = cidx[1::_K]

    expert_in = _dispatch_gather(x, gsrc, E * C, D)
    xe3 = expert_in.reshape(E, C, D)
    b1r = b1.reshape(E, 1, F)
    b2r = b2.reshape(E, 1, D)
    y3 = _expert_ffn(xe3, W1, b1r, W2, b2r, vmask, E, C, D, F)
    y2d = y3.reshape((E + 1) * C, D)
    out = _combine(y2d, cidx0, cidx1, T, D)
    return out.reshape(B, S, D)


# speedup vs baseline: 1.2464x; 1.1842x over previous
import functools

import jax
import jax.numpy as jnp
from jax import lax
from jax.experimental import pallas as pl
from jax.experimental.pallas import tpu as pltpu
from jax.experimental.pallas import tpu_sc as plsc

_K = 2
_CAP = 2
_NC = 2
_NS = 16
_NW = _NC * _NS
_LANES = 16


def _sc_mesh():
    return plsc.VectorSubcoreMesh(
        core_axis_name="c", subcore_axis_name="s", num_cores=_NC,
        num_subcores=_NS)


def _wid():
    return lax.axis_index("s") * _NC + lax.axis_index("c")


def _combine(y2d, cidx0, cidx1, t, d):
    tok_w = t // _NW
    nvc = d // _LANES

    @functools.partial(
        pl.kernel,
        out_type=jax.ShapeDtypeStruct((t, d), jnp.float32),
        mesh=_sc_mesh(),
        scratch_types=[
            pltpu.VMEM((tok_w,), jnp.int32),
            pltpu.VMEM((tok_w,), jnp.int32),
            pltpu.VMEM((tok_w, d), jnp.float32),
            pltpu.VMEM((tok_w, d), jnp.float32),
            pltpu.SemaphoreType.DMA,
            pltpu.SemaphoreType.DMA,
        ],
    )
    def k(y_hbm, c0_hbm, c1_hbm, out_hbm, i0_v, i1_v, g0_v, g1_v, s0, s1):
        base = _wid() * tok_w
        pltpu.sync_copy(c0_hbm.at[pl.ds(base, tok_w)], i0_v)
        cp0 = pltpu.async_copy(y_hbm.at[i0_v], g0_v, s0)
        pltpu.sync_copy(c1_hbm.at[pl.ds(base, tok_w)], i1_v)
        cp1 = pltpu.async_copy(y_hbm.at[i1_v], g1_v, s1)
        cp0.wait()
        cp1.wait()

        def row_add(r, carry):
            for u in range(nvc):
                sl = pl.ds(u * _LANES, _LANES)
                g0_v[r, sl] = g0_v[r, sl] + g1_v[r, sl]
            return carry

        lax.fori_loop(0, tok_w, row_add, 0)
        pltpu.sync_copy(g0_v, out_hbm.at[pl.ds(base, tok_w)])

    return k(y2d, cidx0, cidx1)


def _ffn_body(n_e, n_f, t, src_ref, x_ref, w1_ref, b1_ref, w2_ref, b2_ref,
              m_ref, y_ref, xe_ref, acc_ref):
    e = pl.program_id(0)
    fi = pl.program_id(1)

    @pl.when(e < n_e)
    def _compute():
        c = src_ref.shape[1]

        @pl.when(fi == 0)
        def _gather():
            iota_t = lax.broadcasted_iota(jnp.int32, (c, t), 1)
            onehot = (src_ref[0] == iota_t).astype(jnp.bfloat16)
            xe = jnp.dot(onehot, x_ref[...],
                         preferred_element_type=jnp.float32)
            xe_ref[...] = xe.astype(jnp.bfloat16)

        hm = jnp.dot(xe_ref[...], w1_ref[0].astype(jnp.bfloat16),
                     preferred_element_type=jnp.float32)
        hm = jax.nn.gelu(hm + b1_ref[0, 0, :])
        part = jnp.dot(hm.astype(jnp.bfloat16), w2_ref[0].astype(jnp.bfloat16),
                       preferred_element_type=jnp.float32)

        @pl.when(fi == 0)
        def _():
            acc_ref[...] = part

        @pl.when(fi > 0)
        def _():
            acc_ref[...] += part

        @pl.when(fi == n_f - 1)
        def _():
            y_ref[0] = (acc_ref[...] + b2_ref[0, 0, :]) * m_ref[0]

    @pl.when(e == n_e)
    def _pad_zero():
        y_ref[0] = jnp.zeros_like(y_ref[0])


def _expert_ffn(src_col, x_bf, W1, b1r, W2, b2r, vmask, n_e, c, d, f, t):
    n_f = 2
    fb = f // n_f
    grid = (n_e + 1, n_f)
    clamp = lambda e: jnp.minimum(e, n_e - 1)
    return pl.pallas_call(
        functools.partial(_ffn_body, n_e, n_f, t),
        grid=grid,
        in_specs=[
            pl.BlockSpec((1, c, 1), lambda e, fi: (clamp(e), 0, 0)),
            pl.BlockSpec((t, d), lambda e, fi: (0, 0)),
            pl.BlockSpec((1, d, fb), lambda e, fi: (clamp(e), 0, fi)),
            pl.BlockSpec((1, 1, fb), lambda e, fi: (clamp(e), 0, fi)),
            pl.BlockSpec((1, fb, d), lambda e, fi: (clamp(e), fi, 0)),
            pl.BlockSpec((1, 1, d), lambda e, fi: (clamp(e), 0, 0)),
            pl.BlockSpec((1, c, 1), lambda e, fi: (clamp(e), 0, 0)),
        ],
        out_specs=pl.BlockSpec((1, c, d), lambda e, fi: (e, 0, 0)),
        out_shape=jax.ShapeDtypeStruct((n_e + 1, c, d), jnp.float32),
        scratch_shapes=[
            pltpu.VMEM((c, d), jnp.bfloat16),
            pltpu.VMEM((c, d), jnp.float32),
        ],
    )(src_col, x_bf, W1, b1r, W2, b2r, vmask)


def kernel(hidden_states, hash_proj, W1, b1, W2, b2):
    B, S, D = hidden_states.shape
    E, _, F = W1.shape
    T = B * S
    C = _CAP * (-(-(T * _K) // E))

    x = hidden_states.reshape(T, D)

    h = x @ hash_proj.T
    buckets = jnp.mod(jnp.floor(jnp.abs(h) * 997.0).astype(jnp.int32), E)
    flat_e = buckets[:, :_K].reshape(-1)

    order = jnp.argsort(flat_e, stable=True).astype(jnp.int32)
    counts = jnp.bincount(flat_e, length=E).astype(jnp.int32)
    starts = (jnp.cumsum(counts) - counts).astype(jnp.int32)
    cpos = jnp.arange(C, dtype=jnp.int32)
    slot_j = order[jnp.clip(starts[:, None] + cpos[None, :], 0, T * _K - 1)]
    valid = cpos[None, :] < jnp.minimum(counts, C)[:, None]
    gsrc = jnp.where(valid, slot_j // _K, 0).reshape(-1).astype(jnp.int32)
    vmask = jnp.where(valid, 0.5, 0.0).astype(jnp.float32).reshape(E, C, 1)

    e_sorted = flat_e[order]
    pos_sorted = jnp.arange(T * _K, dtype=jnp.int32) - starts[e_sorted]
    pos = jnp.zeros((T * _K,), jnp.int32).at[order].set(pos_sorted)
    cidx = jnp.where(pos < C, flat_e * C + pos, E * C).astype(jnp.int32)
    cidx0 = cidx[0::_K]
    cidx1 = cidx[1::_K]

    src_col = gsrc.reshape(E, C, 1)
    x_bf = x.astype(jnp.bfloat16)
    b1r = b1.reshape(E, 1, F)
    b2r = b2.reshape(E, 1, D)
    y3 = _expert_ffn(src_col, x_bf, W1, b1r, W2, b2r, vmask, E, C, D, F, T)
    y2d = y3.reshape((E + 1) * C, D)
    out = _combine(y2d, cidx0, cidx1, T, D)
    return out.reshape(B, S, D)


# speedup vs baseline: 1.6546x; 1.3275x over previous
import functools

import jax
import jax.numpy as jnp
from jax import lax
from jax.experimental import pallas as pl
from jax.experimental.pallas import tpu as pltpu
from jax.experimental.pallas import tpu_sc as plsc

_K = 2
_CAP = 2
_NC = 2
_NS = 16
_NW = _NC * _NS
_LANES = 16


def _sc_mesh():
    return plsc.VectorSubcoreMesh(
        core_axis_name="c", subcore_axis_name="s", num_cores=_NC,
        num_subcores=_NS)


def _wid():
    return lax.axis_index("s") * _NC + lax.axis_index("c")


def _combine(y2d, cidx0, cidx1, t, d):
    tok_w = t // _NW
    nvc = d // _LANES

    @functools.partial(
        pl.kernel,
        out_type=jax.ShapeDtypeStruct((t, d), jnp.float32),
        mesh=_sc_mesh(),
        scratch_types=[
            pltpu.VMEM((tok_w,), jnp.int32),
            pltpu.VMEM((tok_w,), jnp.int32),
            pltpu.VMEM((tok_w, d), jnp.float32),
            pltpu.VMEM((tok_w, d), jnp.float32),
            pltpu.SemaphoreType.DMA,
            pltpu.SemaphoreType.DMA,
        ],
    )
    def k(y_hbm, c0_hbm, c1_hbm, out_hbm, i0_v, i1_v, g0_v, g1_v, s0, s1):
        base = _wid() * tok_w
        pltpu.sync_copy(c0_hbm.at[pl.ds(base, tok_w)], i0_v)
        cp0 = pltpu.async_copy(y_hbm.at[i0_v], g0_v, s0)
        pltpu.sync_copy(c1_hbm.at[pl.ds(base, tok_w)], i1_v)
        cp1 = pltpu.async_copy(y_hbm.at[i1_v], g1_v, s1)
        cp0.wait()
        cp1.wait()

        def row_add(r, carry):
            for u in range(nvc):
                sl = pl.ds(u * _LANES, _LANES)
                g0_v[r, sl] = g0_v[r, sl] + g1_v[r, sl]
            return carry

        lax.fori_loop(0, tok_w, row_add, 0)
        pltpu.sync_copy(g0_v, out_hbm.at[pl.ds(base, tok_w)])

    return k(y2d, cidx0, cidx1)


def _ffn_body(n_e, n_f, t, c, e0_ref, e1_ref, p0_ref, p1_ref, x_ref, w1_ref,
              b1_ref, w2_ref, b2_ref, y_ref, xe_ref, acc_ref):
    e = pl.program_id(0)
    fi = pl.program_id(1)

    @pl.when(e < n_e)
    def _compute():
        @pl.when(fi == 0)
        def _gather():
            iota_c = lax.broadcasted_iota(jnp.int32, (c, t), 0)
            oh0 = (p0_ref[...] == iota_c) & (e0_ref[...] == e)
            oh1 = (p1_ref[...] == iota_c) & (e1_ref[...] == e)
            onehot = (oh0.astype(jnp.bfloat16) + oh1.astype(jnp.bfloat16))
            xe = jnp.dot(onehot, x_ref[...],
                         preferred_element_type=jnp.float32)
            xe_ref[...] = xe.astype(jnp.bfloat16)

        hm = jnp.dot(xe_ref[...], w1_ref[0].astype(jnp.bfloat16),
                     preferred_element_type=jnp.float32)
        hm = jax.nn.gelu(hm + b1_ref[0, 0, :])
        part = jnp.dot(hm.astype(jnp.bfloat16), w2_ref[0].astype(jnp.bfloat16),
                       preferred_element_type=jnp.float32)

        @pl.when(fi == 0)
        def _():
            acc_ref[...] = part

        @pl.when(fi > 0)
        def _():
            acc_ref[...] += part

        @pl.when(fi == n_f - 1)
        def _():
            y_ref[0] = (acc_ref[...] + b2_ref[0, 0, :]) * 0.5

    @pl.when(e == n_e)
    def _pad_zero():
        y_ref[0] = jnp.zeros_like(y_ref[0])


def _expert_ffn(e0, e1, p0, p1, x_bf, W1, b1r, W2, b2r, n_e, c, d, f, t):
    n_f = 2
    fb = f // n_f
    grid = (n_e + 1, n_f)
    clamp = lambda e: jnp.minimum(e, n_e - 1)
    row = pl.BlockSpec((1, t), lambda e, fi: (0, 0))
    return pl.pallas_call(
        functools.partial(_ffn_body, n_e, n_f, t, c),
        grid=grid,
        in_specs=[
            row, row, row, row,
            pl.BlockSpec((t, d), lambda e, fi: (0, 0)),
            pl.BlockSpec((1, d, fb), lambda e, fi: (clamp(e), 0, fi)),
            pl.BlockSpec((1, 1, fb), lambda e, fi: (clamp(e), 0, fi)),
            pl.BlockSpec((1, fb, d), lambda e, fi: (clamp(e), fi, 0)),
            pl.BlockSpec((1, 1, d), lambda e, fi: (clamp(e), 0, 0)),
        ],
        out_specs=pl.BlockSpec((1, c, d), lambda e, fi: (e, 0, 0)),
        out_shape=jax.ShapeDtypeStruct((n_e + 1, c, d), jnp.float32),
        scratch_shapes=[
            pltpu.VMEM((c, d), jnp.bfloat16),
            pltpu.VMEM((c, d), jnp.float32),
        ],
    )(e0, e1, p0, p1, x_bf, W1, b1r, W2, b2r)


def kernel(hidden_states, hash_proj, W1, b1, W2, b2):
    B, S, D = hidden_states.shape
    E, _, F = W1.shape
    T = B * S
    C = _CAP * (-(-(T * _K) // E))

    x = hidden_states.reshape(T, D)

    h = x @ hash_proj.T
    buckets = jnp.mod(jnp.floor(jnp.abs(h) * 997.0).astype(jnp.int32), E)
    flat_e = buckets[:, :_K].reshape(-1)

    onehot_j = (flat_e[:, None] == jnp.arange(E, dtype=jnp.int32)[None, :])
    onehot_j = onehot_j.astype(jnp.int32)
    pos = jnp.sum(jnp.cumsum(onehot_j, axis=0) * onehot_j, axis=-1) - 1

    ep = flat_e.reshape(T, _K)
    pp = pos.reshape(T, _K)
    e0 = ep[:, 0].reshape(1, T)
    e1 = ep[:, 1].reshape(1, T)
    p0 = pp[:, 0].reshape(1, T)
    p1 = pp[:, 1].reshape(1, T)

    cidx = jnp.where(pos < C, flat_e * C + pos, E * C).astype(jnp.int32)
    cidx2 = cidx.reshape(T, _K)
    cidx0 = cidx2[:, 0]
    cidx1 = cidx2[:, 1]

    x_bf = x.astype(jnp.bfloat16)
    b1r = b1.reshape(E, 1, F)
    b2r = b2.reshape(E, 1, D)
    y3 = _expert_ffn(e0, e1, p0, p1, x_bf, W1, b1r, W2, b2r, E, C, D, F, T)
    y2d = y3.reshape((E + 1) * C, D)
    out = _combine(y2d, cidx0, cidx1, T, D)
    return out.reshape(B, S, D)


# speedup vs baseline: 1.9576x; 1.1831x over previous
import functools

import jax
import jax.numpy as jnp
from jax import lax
from jax.experimental import pallas as pl
from jax.experimental.pallas import tpu as pltpu
from jax.experimental.pallas import tpu_sc as plsc

_K = 2
_CAP = 2
_NC = 2
_NS = 16
_NW = _NC * _NS
_LANES = 16


def _sc_mesh():
    return plsc.VectorSubcoreMesh(
        core_axis_name="c", subcore_axis_name="s", num_cores=_NC,
        num_subcores=_NS)


def _wid():
    return lax.axis_index("s") * _NC + lax.axis_index("c")


def _combine(y2d, cidx0, cidx1, t, d):
    tok_w = t // _NW
    nvc = d // _LANES

    @functools.partial(
        pl.kernel,
        out_type=jax.ShapeDtypeStruct((t, d), jnp.float32),
        mesh=_sc_mesh(),
        scratch_types=[
            pltpu.VMEM((tok_w,), jnp.int32),
            pltpu.VMEM((tok_w,), jnp.int32),
            pltpu.VMEM((tok_w, d), jnp.float32),
            pltpu.VMEM((tok_w, d), jnp.float32),
            pltpu.SemaphoreType.DMA,
            pltpu.SemaphoreType.DMA,
        ],
    )
    def k(y_hbm, c0_hbm, c1_hbm, out_hbm, i0_v, i1_v, g0_v, g1_v, s0, s1):
        base = _wid() * tok_w
        pltpu.sync_copy(c0_hbm.at[pl.ds(base, tok_w)], i0_v)
        cp0 = pltpu.async_copy(y_hbm.at[i0_v], g0_v, s0)
        pltpu.sync_copy(c1_hbm.at[pl.ds(base, tok_w)], i1_v)
        cp1 = pltpu.async_copy(y_hbm.at[i1_v], g1_v, s1)
        cp0.wait()
        cp1.wait()

        def row_add(r, carry):
            for u in range(nvc):
                sl = pl.ds(u * _LANES, _LANES)
                g0_v[r, sl] = g0_v[r, sl] + g1_v[r, sl]
            return carry

        lax.fori_loop(0, tok_w, row_add, 0)
        pltpu.sync_copy(g0_v, out_hbm.at[pl.ds(base, tok_w)])

    return k(y2d, cidx0, cidx1)


def _ffn_body(n_e, n_f, t, c, e0_ref, e1_ref, p0_ref, p1_ref, x_ref, w1_ref,
              b1_ref, w2_ref, b2_ref, y_ref, xe_ref, acc_ref):
    e = pl.program_id(0)
    fi = pl.program_id(1)

    @pl.when(e < n_e)
    def _compute():
        @pl.when(fi == 0)
        def _gather():
            iota_c = lax.broadcasted_iota(jnp.int32, (c, t), 0)
            oh0 = (p0_ref[...] == iota_c) & (e0_ref[...] == e)
            oh1 = (p1_ref[...] == iota_c) & (e1_ref[...] == e)
            onehot = (oh0.astype(jnp.bfloat16) + oh1.astype(jnp.bfloat16))
            xe = jnp.dot(onehot, x_ref[...],
                         preferred_element_type=jnp.float32)
            xe_ref[...] = xe.astype(jnp.bfloat16)

        hm = jnp.dot(xe_ref[...], w1_ref[0].astype(jnp.bfloat16),
                     preferred_element_type=jnp.float32)
        hm = jax.nn.gelu(hm + b1_ref[0, 0, :])
        part = jnp.dot(hm.astype(jnp.bfloat16), w2_ref[0].astype(jnp.bfloat16),
                       preferred_element_type=jnp.float32)

        @pl.when(fi == 0)
        def _():
            acc_ref[...] = part

        @pl.when(fi > 0)
        def _():
            acc_ref[...] += part

        @pl.when(fi == n_f - 1)
        def _():
            y_ref[0] = (acc_ref[...] + b2_ref[0, 0, :]) * 0.5

    @pl.when(e == n_e)
    def _pad_zero():
        y_ref[0] = jnp.zeros_like(y_ref[0])


def _expert_ffn(e0, e1, p0, p1, x_bf, W1, b1r, W2, b2r, n_e, c, d, f, t):
    n_f = 1
    fb = f // n_f
    grid = (n_e + 1, n_f)
    clamp = lambda e: jnp.minimum(e, n_e - 1)
    row = pl.BlockSpec((1, t), lambda e, fi: (0, 0))
    return pl.pallas_call(
        functools.partial(_ffn_body, n_e, n_f, t, c),
        grid=grid,
        in_specs=[
            row, row, row, row,
            pl.BlockSpec((t, d), lambda e, fi: (0, 0)),
            pl.BlockSpec((1, d, fb), lambda e, fi: (clamp(e), 0, fi)),
            pl.BlockSpec((1, 1, fb), lambda e, fi: (clamp(e), 0, fi)),
            pl.BlockSpec((1, fb, d), lambda e, fi: (clamp(e), fi, 0)),
            pl.BlockSpec((1, 1, d), lambda e, fi: (clamp(e), 0, 0)),
        ],
        out_specs=pl.BlockSpec((1, c, d), lambda e, fi: (e, 0, 0)),
        out_shape=jax.ShapeDtypeStruct((n_e + 1, c, d), jnp.float32),
        scratch_shapes=[
            pltpu.VMEM((c, d), jnp.bfloat16),
            pltpu.VMEM((c, d), jnp.float32),
        ],
    )(e0, e1, p0, p1, x_bf, W1, b1r, W2, b2r)


def kernel(hidden_states, hash_proj, W1, b1, W2, b2):
    B, S, D = hidden_states.shape
    E, _, F = W1.shape
    T = B * S
    C = _CAP * (-(-(T * _K) // E))

    x = hidden_states.reshape(T, D)

    h = x @ hash_proj.T
    buckets = jnp.mod(jnp.floor(jnp.abs(h) * 997.0).astype(jnp.int32), E)
    flat_e = buckets[:, :_K].reshape(-1)

    onehot_j = (flat_e[:, None] == jnp.arange(E, dtype=jnp.int32)[None, :])
    onehot_j = onehot_j.astype(jnp.int32)
    pos = jnp.sum(jnp.cumsum(onehot_j, axis=0) * onehot_j, axis=-1) - 1

    ep = flat_e.reshape(T, _K)
    pp = pos.reshape(T, _K)
    e0 = ep[:, 0].reshape(1, T)
    e1 = ep[:, 1].reshape(1, T)
    p0 = pp[:, 0].reshape(1, T)
    p1 = pp[:, 1].reshape(1, T)

    cidx = jnp.where(pos < C, flat_e * C + pos, E * C).astype(jnp.int32)
    cidx2 = cidx.reshape(T, _K)
    cidx0 = cidx2[:, 0]
    cidx1 = cidx2[:, 1]

    x_bf = x.astype(jnp.bfloat16)
    b1r = b1.reshape(E, 1, F)
    b2r = b2.reshape(E, 1, D)
    y3 = _expert_ffn(e0, e1, p0, p1, x_bf, W1, b1r, W2, b2r, E, C, D, F, T)
    y2d = y3.reshape((E + 1) * C, D)
    out = _combine(y2d, cidx0, cidx1, T, D)
    return out.reshape(B, S, D)
